# baseline (speedup 1.0000x reference)
import functools

import jax
import jax.numpy as jnp
from jax import lax
from jax.experimental import pallas as pl
from jax.experimental.layout import Format, Layout, with_layout_constraint
from jax.experimental.pallas import tpu as pltpu
from jax.experimental.pallas import tpu_sc as plsc

NUM_EMBEDDINGS = 1000000
EMBEDDING_DIM = 64
BATCH = 16384
N_FIELDS = 26

NC = 2
NS = 16
NW = NC * NS

B = BATCH * N_FIELDS
GB = 128
CH = 2
G = GB * CH
J_PER_W = B // NW
N_UNITS = J_PER_W // G
NBUF = 3
LANES = 16
B_PER_W = BATCH // NW


def _mesh():
    return plsc.VectorSubcoreMesh(
        core_axis_name="c", subcore_axis_name="s",
        num_cores=NC, num_subcores=NS,
    )


def _flatten_body(xt_hbm, idx_hbm, stage_v, flat_v, sem_in):
    wid = lax.axis_index("s") * NC + lax.axis_index("c")
    b_base = wid * B_PER_W

    def in_descs():
        return [
            (xt_hbm.at[f, pl.ds(b_base + c * GB, GB)],
             stage_v.at[f, pl.ds(c * GB, GB)])
            for f in range(N_FIELDS)
            for c in range(B_PER_W // GB)
        ]

    for src, dst in in_descs():
        pltpu.async_copy(src, dst, sem_in)
    for src, dst in in_descs():
        pltpu.make_async_copy(src, dst, sem_in).wait()

    iota = lax.iota(jnp.int32, LANES)

    def group(g, carry):
        b0, f0 = carry
        f_raw = jnp.full((LANES,), f0, jnp.int32) + iota
        wrap = (f_raw >= N_FIELDS).astype(jnp.int32)
        f_vec = f_raw - N_FIELDS * wrap
        b_vec = jnp.full((LANES,), b0, jnp.int32) + wrap
        vals = plsc.load_gather(stage_v, [f_vec, b_vec])
        flat_v[pl.ds(g * LANES, LANES)] = vals
        nf = f0 + LANES
        carry_wrap = (nf >= N_FIELDS).astype(jnp.int32)
        return (b0 + carry_wrap, nf - N_FIELDS * carry_wrap)

    lax.fori_loop(0, J_PER_W // LANES, group,
                  (jnp.int32(0), jnp.int32(0)))
    pltpu.sync_copy(flat_v, idx_hbm.at[pl.ds(wid * J_PER_W, J_PER_W)])


def _flatten_indices(x):
    xt = x.astype(jnp.int32).T
    return pl.kernel(
        _flatten_body,
        out_type=jax.ShapeDtypeStruct((B,), jnp.int32),
        mesh=_mesh(),
        scratch_types=[
            pltpu.VMEM((N_FIELDS, B_PER_W), jnp.int32),
            pltpu.VMEM((J_PER_W,), jnp.int32),
            pltpu.SemaphoreType.DMA,
        ],
        compiler_params=pltpu.CompilerParams(
            use_tc_tiling_on_sc=True, needs_layout_passes=False),
    )(xt)


VCH = 128
N_VCH = NUM_EMBEDDINGS // VCH
P_ROWS = NUM_EMBEDDINGS // 2


def _pack_body(wt_hbm, p_hbm, in_v, out_v, sem_i, sem_o):
    wid = lax.axis_index("s") * NC + lax.axis_index("c")
    n_base = 244 * wid + lax.min(wid, 4)
    n_mine = 244 + (wid < 4).astype(jnp.int32)
    iota = lax.iota(jnp.int32, LANES)

    def fire(c, s):
        pltpu.async_copy(wt_hbm.at[:, pl.ds(c * VCH, VCH)], in_v.at[s],
                         sem_i)

    def wait_in(s):
        pltpu.make_async_copy(wt_hbm.at[:, pl.ds(0, VCH)], in_v.at[s],
                              sem_i).wait()

    def transpose(s, nrows):
        def vrow(v, carry):
            row = lax.shift_right_logical(v, 1)
            colb = lax.rem(v, 2) * EMBEDDING_DIM
            for g in range(EMBEDDING_DIM // LANES):
                vals = plsc.load_gather(
                    in_v.at[s], [iota + g * LANES,
                                 jnp.full((LANES,), v, jnp.int32)])
                out_v[s, row, pl.ds(colb + g * LANES, LANES)] = vals
            return carry

        lax.fori_loop(0, nrows, vrow, 0)

    def writeback(c, s):
        pltpu.async_copy(out_v.at[s], p_hbm.at[pl.ds(c * (VCH // 2),
                                                     VCH // 2)], sem_o)

    def wait_writeback(s):
        pltpu.make_async_copy(out_v.at[s],
                              p_hbm.at[pl.ds(0, VCH // 2)], sem_o).wait()

    def chunk(i, carry):
        c = n_base + i

        def do(s):
            fire(c, s)
            wait_in(s)

            @pl.when(i >= 2)
            def _():
                wait_writeback(s)

            transpose(s, VCH)
            writeback(c, s)

        @pl.when(lax.rem(i, 2) == 0)
        def _():
            do(0)

        @pl.when(lax.rem(i, 2) == 1)
        def _():
            do(1)

        return carry

    lax.fori_loop(0, n_mine, chunk, 0)
    wait_writeback(0)
    wait_writeback(1)

    @pl.when(wid == 0)
    def _tail():
        for f in range(EMBEDDING_DIM):
            pltpu.async_copy(
                wt_hbm.at[f, pl.ds(N_VCH * VCH, EMBEDDING_DIM)],
                in_v.at[0].at[f, pl.ds(0, EMBEDDING_DIM)], sem_i)
        for f in range(EMBEDDING_DIM):
            pltpu.make_async_copy(
                wt_hbm.at[f, pl.ds(N_VCH * VCH, EMBEDDING_DIM)],
                in_v.at[0].at[f, pl.ds(0, EMBEDDING_DIM)], sem_i).wait()
        transpose(0, EMBEDDING_DIM)
        pltpu.sync_copy(out_v.at[0].at[pl.ds(0, EMBEDDING_DIM // 2)],
                        p_hbm.at[pl.ds(N_VCH * (VCH // 2),
                                       EMBEDDING_DIM // 2)])


def _pack_table(embedding_weight):
    wt = embedding_weight.T
    packed = pl.kernel(
        _pack_body,
        out_type=jax.ShapeDtypeStruct((P_ROWS, 2 * EMBEDDING_DIM),
                                      jnp.float32),
        mesh=_mesh(),
        scratch_types=[
            pltpu.VMEM((2, EMBEDDING_DIM, VCH), jnp.float32),
            pltpu.VMEM((2, VCH // 2, 2 * EMBEDDING_DIM), jnp.float32),
            pltpu.SemaphoreType.DMA,
            pltpu.SemaphoreType.DMA,
        ],
        compiler_params=pltpu.CompilerParams(
            use_tc_tiling_on_sc=True, needs_layout_passes=False),
    )(wt)
    return packed.reshape(NUM_EMBEDDINGS, EMBEDDING_DIM)


def _gather_body(idx_hbm, table_hbm, out_hbm, idx_v, rows_v,
                 sg0, sg1, sg2, so0, so1, so2):
    sem_g = [sg0, sg1, sg2]
    sem_o = [so0, so1, so2]
    wid = lax.axis_index("s") * NC + lax.axis_index("c")
    j_base = wid * J_PER_W

    def fire(k, s):
        j0 = j_base + k * G
        for j in range(CH):
            pltpu.sync_copy(idx_hbm.at[pl.ds(j0 + j * GB, GB)],
                            idx_v.at[s].at[j])
        for j in range(CH):
            pltpu.async_copy(
                table_hbm.at[idx_v.at[s].at[j]],
                rows_v.at[s].at[pl.ds(j * GB, GB)],
                sem_g[s],
            )

    def wait_gathers(s):
        for j in range(CH):
            pltpu.make_async_copy(
                table_hbm.at[idx_v.at[s].at[j]],
                rows_v.at[s].at[pl.ds(j * GB, GB)],
                sem_g[s],
            ).wait()

    def writeback(k, s):
        j0 = j_base + k * G
        pltpu.async_copy(rows_v.at[s], out_hbm.at[pl.ds(j0, G)], sem_o[s])

    def wait_writeback(k, s):
        j0 = j_base + k * G
        pltpu.make_async_copy(
            rows_v.at[s], out_hbm.at[pl.ds(j0, G)], sem_o[s]
        ).wait()

    fire(0, 0)
    fire(1, 1)
    wait_gathers(0)
    writeback(0, 0)
    fire(2, 2)
    wait_gathers(1)
    writeback(1, 1)

    def triple(t, carry):
        for j in range(NBUF):
            k = 3 * t + j
            s = j
            sp = (j + NBUF - 1) % NBUF
            wait_writeback(k - NBUF, s)
            fire(k, s)
            wait_gathers(sp)
            writeback(k - 1, sp)
        return carry

    lax.fori_loop(1, 17, triple, 0)

    k = N_UNITS - 1
    wait_writeback(k - NBUF, 0)
    fire(k, 0)
    wait_gathers(2)
    writeback(k - 1, 2)
    wait_gathers(0)
    writeback(k, 0)
    wait_writeback(N_UNITS - 3, 1)
    wait_writeback(N_UNITS - 2, 2)
    wait_writeback(N_UNITS - 1, 0)


@functools.partial(jax.jit, static_argnames=())
def kernel(x, embedding_weight):
    idx_flat = _flatten_indices(x)
    table = _pack_table(embedding_weight)
    out = pl.kernel(
        _gather_body,
        out_type=jax.ShapeDtypeStruct((B, EMBEDDING_DIM), jnp.float32),
        mesh=_mesh(),
        scratch_types=[
            pltpu.VMEM((NBUF, CH, GB), jnp.int32),
            pltpu.VMEM((NBUF, G, EMBEDDING_DIM), jnp.float32),
            pltpu.SemaphoreType.DMA,
            pltpu.SemaphoreType.DMA,
            pltpu.SemaphoreType.DMA,
            pltpu.SemaphoreType.DMA,
            pltpu.SemaphoreType.DMA,
            pltpu.SemaphoreType.DMA,
        ],
        compiler_params=pltpu.CompilerParams(
            use_tc_tiling_on_sc=False, needs_layout_passes=False),
    )(idx_flat, table)
    out3 = with_layout_constraint(
        out.reshape(BATCH, N_FIELDS, EMBEDDING_DIM),
        Layout(major_to_minor=(0, 1, 2), tiling=((8,),)),
    )
    return out3


# speedup vs baseline: 2.2032x; 2.2032x over previous
import functools

import jax
import jax.numpy as jnp
from jax import lax
from jax.experimental import pallas as pl
from jax.experimental.layout import Format, Layout, with_layout_constraint
from jax.experimental.pallas import tpu as pltpu
from jax.experimental.pallas import tpu_sc as plsc

NUM_EMBEDDINGS = 1000000
EMBEDDING_DIM = 64
BATCH = 16384
N_FIELDS = 26

NC = 2
NS = 16
NW = NC * NS

B = BATCH * N_FIELDS
GB = 128
CH = 2
G = GB * CH
J_PER_W = B // NW
N_UNITS = J_PER_W // G
NBUF = 3
LANES = 16
B_PER_W = BATCH // NW


def _mesh():
    return plsc.VectorSubcoreMesh(
        core_axis_name="c", subcore_axis_name="s",
        num_cores=NC, num_subcores=NS,
    )


def _flatten_body(xt_hbm, idx_hbm, stage_v, flat_v, sem_in):
    wid = lax.axis_index("s") * NC + lax.axis_index("c")
    b_base = wid * B_PER_W

    def in_descs():
        return [
            (xt_hbm.at[f, pl.ds(b_base + c * GB, GB)],
             stage_v.at[f, pl.ds(c * GB, GB)])
            for f in range(N_FIELDS)
            for c in range(B_PER_W // GB)
        ]

    for src, dst in in_descs():
        pltpu.async_copy(src, dst, sem_in)
    for src, dst in in_descs():
        pltpu.make_async_copy(src, dst, sem_in).wait()

    iota = lax.iota(jnp.int32, LANES)

    def group(g, carry):
        b0, f0 = carry
        f_raw = jnp.full((LANES,), f0, jnp.int32) + iota
        wrap = (f_raw >= N_FIELDS).astype(jnp.int32)
        f_vec = f_raw - N_FIELDS * wrap
        b_vec = jnp.full((LANES,), b0, jnp.int32) + wrap
        vals = plsc.load_gather(stage_v, [f_vec, b_vec])
        flat_v[pl.ds(g * LANES, LANES)] = vals
        nf = f0 + LANES
        carry_wrap = (nf >= N_FIELDS).astype(jnp.int32)
        return (b0 + carry_wrap, nf - N_FIELDS * carry_wrap)

    lax.fori_loop(0, J_PER_W // LANES, group,
                  (jnp.int32(0), jnp.int32(0)))
    pltpu.sync_copy(flat_v, idx_hbm.at[pl.ds(wid * J_PER_W, J_PER_W)])


def _flatten_indices(x):
    xt = x.astype(jnp.int32).T
    return pl.kernel(
        _flatten_body,
        out_type=jax.ShapeDtypeStruct((B,), jnp.int32),
        mesh=_mesh(),
        scratch_types=[
            pltpu.VMEM((N_FIELDS, B_PER_W), jnp.int32),
            pltpu.VMEM((J_PER_W,), jnp.int32),
            pltpu.SemaphoreType.DMA,
        ],
        compiler_params=pltpu.CompilerParams(
            use_tc_tiling_on_sc=True, needs_layout_passes=False),
    )(xt)


def _gather_body(idx_hbm, table_hbm, out_hbm, idx_v, rows_v,
                 sg0, sg1, sg2, so0, so1, so2):
    sem_g = [sg0, sg1, sg2]
    sem_o = [so0, so1, so2]
    wid = lax.axis_index("s") * NC + lax.axis_index("c")
    j_base = wid * J_PER_W

    def fire(k, s):
        j0 = j_base + k * G
        for j in range(CH):
            pltpu.sync_copy(idx_hbm.at[pl.ds(j0 + j * GB, GB)],
                            idx_v.at[s].at[j])
        for j in range(CH):
            pltpu.async_copy(
                table_hbm.at[idx_v.at[s].at[j]],
                rows_v.at[s].at[pl.ds(j * GB, GB)],
                sem_g[s],
            )

    def wait_gathers(s):
        for j in range(CH):
            pltpu.make_async_copy(
                table_hbm.at[idx_v.at[s].at[j]],
                rows_v.at[s].at[pl.ds(j * GB, GB)],
                sem_g[s],
            ).wait()

    def writeback(k, s):
        j0 = j_base + k * G
        pltpu.async_copy(rows_v.at[s], out_hbm.at[pl.ds(j0, G)], sem_o[s])

    def wait_writeback(k, s):
        j0 = j_base + k * G
        pltpu.make_async_copy(
            rows_v.at[s], out_hbm.at[pl.ds(j0, G)], sem_o[s]
        ).wait()

    fire(0, 0)
    fire(1, 1)
    wait_gathers(0)
    writeback(0, 0)
    fire(2, 2)
    wait_gathers(1)
    writeback(1, 1)

    def triple(t, carry):
        for j in range(NBUF):
            k = 3 * t + j
            s = j
            sp = (j + NBUF - 1) % NBUF
            wait_writeback(k - NBUF, s)
            fire(k, s)
            wait_gathers(sp)
            writeback(k - 1, sp)
        return carry

    lax.fori_loop(1, 17, triple, 0)

    k = N_UNITS - 1
    wait_writeback(k - NBUF, 0)
    fire(k, 0)
    wait_gathers(2)
    writeback(k - 1, 2)
    wait_gathers(0)
    writeback(k, 0)
    wait_writeback(N_UNITS - 3, 1)
    wait_writeback(N_UNITS - 2, 2)
    wait_writeback(N_UNITS - 1, 0)


@functools.partial(jax.jit, static_argnames=())
def kernel(x, embedding_weight):
    idx_flat = _flatten_indices(x)
    table = embedding_weight
    out = pl.kernel(
        _gather_body,
        out_type=jax.ShapeDtypeStruct((B, EMBEDDING_DIM), jnp.float32),
        mesh=_mesh(),
        scratch_types=[
            pltpu.VMEM((NBUF, CH, GB), jnp.int32),
            pltpu.VMEM((NBUF, G, EMBEDDING_DIM), jnp.float32),
            pltpu.SemaphoreType.DMA,
            pltpu.SemaphoreType.DMA,
            pltpu.SemaphoreType.DMA,
            pltpu.SemaphoreType.DMA,
            pltpu.SemaphoreType.DMA,
            pltpu.SemaphoreType.DMA,
        ],
        compiler_params=pltpu.CompilerParams(
            use_tc_tiling_on_sc=False, needs_layout_passes=False),
    )(idx_flat, table)
    out3 = with_layout_constraint(
        out.reshape(BATCH, N_FIELDS, EMBEDDING_DIM),
        Layout(major_to_minor=(0, 1, 2), tiling=((8,),)),
    )
    return out3
